# baseline (speedup 1.0000x reference)
import functools

import jax
import jax.numpy as jnp
import numpy as np
from jax import lax
from jax.experimental import pallas as pl
from jax.experimental.pallas import tpu as pltpu
from jax.experimental.pallas import tpu_sc as plsc

_F = 18
_S = _F - 2
_D = 128
_B = 1024

_NC = 2
_NS = 16
_NW = _NC * _NS
_NPW = _B // _NW
_RPW = _NPW * _S
_IDX_ROWS = _RPW // 128

_NODE_BLK = 64
_COL_BLK = _NODE_BLK * _D


def _make_P() -> np.ndarray:
    A = np.zeros((_F, _F))
    Dg = np.eye(_F)
    Dg[0, 0] = (_F - 1) ** (-0.5)
    for i in range(_F - 1):
        A[0, i + 1] = 1.0
        A[i + 1, 0] = 1.0
    filt = np.eye(_F) - Dg @ (A @ Dg)
    _, U = np.linalg.eigh(filt)
    return U.astype(np.float32)[1:_S + 1, :]


_P_CONST = _make_P()


def _sc_gather_body(neigh_hbm, table_hbm, rows_hbm,
                    idx_v, rows_v, sem_g, sem_w):
    wid = lax.axis_index("s") * _NC + lax.axis_index("c")

    pltpu.sync_copy(neigh_hbm.at[pl.ds(wid * _IDX_ROWS, _IDX_ROWS)], idx_v)

    gets = [
        pltpu.async_copy(table_hbm.at[idx_v.at[q]],
                         rows_v.at[pl.ds(q * 128, 128)], sem_g)
        for q in range(_IDX_ROWS)
    ]

    def _put(t):
        return pltpu.async_copy(
            rows_v.at[pl.ds(t * _NPW, _NPW)].reshape(1, _NPW * _D),
            rows_hbm.at[pl.ds(t, 1), pl.ds(wid * _NPW * _D, _NPW * _D)],
            sem_w)

    gets[0].wait()
    gets[1].wait()
    puts = [_put(t) for t in range(8)]
    gets[2].wait()
    gets[3].wait()
    puts += [_put(t) for t in range(8, _S)]
    for c in puts:
        c.wait()


@functools.partial(
    pl.kernel,
    out_type=jax.ShapeDtypeStruct((_S, _B * _D), jnp.float32),
    mesh=plsc.VectorSubcoreMesh(core_axis_name="c", subcore_axis_name="s"),
    scratch_types=[
        pltpu.VMEM((_IDX_ROWS, 128), jnp.int32),
        pltpu.VMEM((_RPW, _D), jnp.float32),
        pltpu.SemaphoreType.DMA,
        pltpu.SemaphoreType.DMA,
    ],
)
def _sc_gather(neigh_hbm, table_hbm, rows_hbm,
               idx_v, rows_v, sem_g, sem_w):
    _sc_gather_body(neigh_hbm, table_hbm, rows_hbm,
                    idx_v, rows_v, sem_g, sem_w)


def _spectral_body(x_ref, p_ref, w_ref, m_ref, o_ref):
    xm = x_ref[...]
    p = p_ref[...]
    w2 = lax.dot_general(p, xm, (((0,), (0,)), ((), ())),
                         precision=lax.Precision.DEFAULT,
                         preferred_element_type=jnp.float32)
    m3 = w_ref[...] * w2
    tf = lax.dot_general(p, m3, (((1,), (0,)), ((), ())),
                         precision=lax.Precision.DEFAULT,
                         preferred_element_type=jnp.float32)
    o_ref[...] = lax.dot_general(m_ref[...], tf, (((1,), (0,)), ((), ())),
                                 precision=lax.Precision.DEFAULT,
                                 preferred_element_type=jnp.float32)


def _bn_body(x_ref, gamma_ref, beta_ref, o_ref):
    x = x_ref[...]
    mean = jnp.mean(x, axis=0, keepdims=True)
    xc = x - mean
    var = jnp.mean(xc * xc, axis=0, keepdims=True)
    inv = lax.rsqrt(var + 1e-5)
    o_ref[...] = xc * (inv * gamma_ref[...]) + beta_ref[...]


def kernel(nodes, batch_neighbors, feat_table, weight, bn_gamma, bn_beta):
    del nodes
    neigh = (batch_neighbors.reshape(_NW, _NPW, _S)
             .transpose(0, 2, 1)
             .reshape(_NW * _IDX_ROWS, 128))
    x = _sc_gather(neigh, feat_table)

    bf_flat = pl.pallas_call(
        _spectral_body,
        grid=(_B // _NODE_BLK,),
        in_specs=[
            pl.BlockSpec((_S, _COL_BLK), lambda i: (0, i)),
            pl.BlockSpec((_S, _F), lambda i: (0, 0)),
            pl.BlockSpec((_F, 1), lambda i: (0, 0)),
            pl.BlockSpec((1, _S), lambda i: (0, 0)),
        ],
        out_specs=pl.BlockSpec((1, _COL_BLK), lambda i: (0, i)),
        out_shape=jax.ShapeDtypeStruct((1, _B * _D), jnp.float32),
    )(x, jnp.asarray(_P_CONST), weight,
      jnp.full((1, _S), 1.0 / _S, jnp.float32))

    bf = bf_flat.reshape(_B, _D)
    return pl.pallas_call(
        _bn_body,
        out_shape=jax.ShapeDtypeStruct((_B, _D), jnp.float32),
    )(bf, bn_gamma.reshape(1, _D), bn_beta.reshape(1, _D))


# speedup vs baseline: 1.1260x; 1.1260x over previous
import functools

import jax
import jax.numpy as jnp
import numpy as np
from jax import lax
from jax.experimental import pallas as pl
from jax.experimental.pallas import tpu as pltpu
from jax.experimental.pallas import tpu_sc as plsc

_F = 18
_S = _F - 2
_D = 128
_B = 1024

_NC = 2
_NS = 16
_NW = _NC * _NS
_NPW = _B // _NW
_RPW = _NPW * _S
_IDX_ROWS = _RPW // 128

_NODE_BLK = 128
_COL_BLK = _NODE_BLK * _D


def _make_P() -> np.ndarray:
    A = np.zeros((_F, _F))
    Dg = np.eye(_F)
    Dg[0, 0] = (_F - 1) ** (-0.5)
    for i in range(_F - 1):
        A[0, i + 1] = 1.0
        A[i + 1, 0] = 1.0
    filt = np.eye(_F) - Dg @ (A @ Dg)
    _, U = np.linalg.eigh(filt)
    return U.astype(np.float32)[1:_S + 1, :]


_P_CONST = _make_P()


def _sc_gather_body(neigh_hbm, table_hbm, rows_hbm,
                    idx_v, rows_v, sem_g, sem_w):
    wid = lax.axis_index("s") * _NC + lax.axis_index("c")

    pltpu.sync_copy(neigh_hbm.at[pl.ds(wid * _IDX_ROWS, _IDX_ROWS)], idx_v)

    gets = [
        pltpu.async_copy(table_hbm.at[idx_v.at[q]],
                         rows_v.at[pl.ds(q * 128, 128)], sem_g)
        for q in range(_IDX_ROWS)
    ]

    def _put(t):
        return pltpu.async_copy(
            rows_v.at[pl.ds(t * _NPW, _NPW)].reshape(1, _NPW * _D),
            rows_hbm.at[pl.ds(t, 1), pl.ds(wid * _NPW * _D, _NPW * _D)],
            sem_w)

    gets[0].wait()
    gets[1].wait()
    puts = [_put(t) for t in range(8)]
    gets[2].wait()
    gets[3].wait()
    puts += [_put(t) for t in range(8, _S)]
    for c in puts:
        c.wait()


@functools.partial(
    pl.kernel,
    out_type=jax.ShapeDtypeStruct((_S, _B * _D), jnp.float32),
    mesh=plsc.VectorSubcoreMesh(core_axis_name="c", subcore_axis_name="s"),
    scratch_types=[
        pltpu.VMEM((_IDX_ROWS, 128), jnp.int32),
        pltpu.VMEM((_RPW, _D), jnp.float32),
        pltpu.SemaphoreType.DMA,
        pltpu.SemaphoreType.DMA,
    ],
)
def _sc_gather(neigh_hbm, table_hbm, rows_hbm,
               idx_v, rows_v, sem_g, sem_w):
    _sc_gather_body(neigh_hbm, table_hbm, rows_hbm,
                    idx_v, rows_v, sem_g, sem_w)


def _spectral_body(x_ref, p_ref, w_ref, m_ref, o_ref):
    xm = x_ref[...]
    p = p_ref[...]
    w2 = lax.dot_general(p, xm, (((0,), (0,)), ((), ())),
                         precision=lax.Precision.DEFAULT,
                         preferred_element_type=jnp.float32)
    m3 = w_ref[...] * w2
    tf = lax.dot_general(p, m3, (((1,), (0,)), ((), ())),
                         precision=lax.Precision.DEFAULT,
                         preferred_element_type=jnp.float32)
    o_ref[...] = lax.dot_general(m_ref[...], tf, (((1,), (0,)), ((), ())),
                                 precision=lax.Precision.DEFAULT,
                                 preferred_element_type=jnp.float32)


def _bn_body(x_ref, gamma_ref, beta_ref, o_ref):
    x = x_ref[...]
    mean = jnp.mean(x, axis=0, keepdims=True)
    xc = x - mean
    var = jnp.mean(xc * xc, axis=0, keepdims=True)
    inv = lax.rsqrt(var + 1e-5)
    o_ref[...] = xc * (inv * gamma_ref[...]) + beta_ref[...]


def kernel(nodes, batch_neighbors, feat_table, weight, bn_gamma, bn_beta):
    del nodes
    neigh = (batch_neighbors.reshape(_NW, _NPW, _S)
             .transpose(0, 2, 1)
             .reshape(_NW * _IDX_ROWS, 128))
    x = _sc_gather(neigh, feat_table)

    bf_flat = pl.pallas_call(
        _spectral_body,
        grid=(_B // _NODE_BLK,),
        in_specs=[
            pl.BlockSpec((_S, _COL_BLK), lambda i: (0, i)),
            pl.BlockSpec((_S, _F), lambda i: (0, 0)),
            pl.BlockSpec((_F, 1), lambda i: (0, 0)),
            pl.BlockSpec((1, _S), lambda i: (0, 0)),
        ],
        out_specs=pl.BlockSpec((1, _COL_BLK), lambda i: (0, i)),
        out_shape=jax.ShapeDtypeStruct((1, _B * _D), jnp.float32),
    )(x, jnp.asarray(_P_CONST), weight,
      jnp.full((1, _S), 1.0 / _S, jnp.float32))

    bf = bf_flat.reshape(_B, _D)
    return pl.pallas_call(
        _bn_body,
        out_shape=jax.ShapeDtypeStruct((_B, _D), jnp.float32),
    )(bf, bn_gamma.reshape(1, _D), bn_beta.reshape(1, _D))


# speedup vs baseline: 1.1271x; 1.0010x over previous
import functools

import jax
import jax.numpy as jnp
import numpy as np
from jax import lax
from jax.experimental import pallas as pl
from jax.experimental.pallas import tpu as pltpu
from jax.experimental.pallas import tpu_sc as plsc

_F = 18
_S = _F - 2
_D = 128
_B = 1024

_NC = 2
_NS = 16
_NW = _NC * _NS
_NPW = _B // _NW
_RPW = _NPW * _S
_IDX_ROWS = _RPW // 128

_NODE_BLK = 128
_COL_BLK = _NODE_BLK * _D


def _make_P() -> np.ndarray:
    A = np.zeros((_F, _F))
    Dg = np.eye(_F)
    Dg[0, 0] = (_F - 1) ** (-0.5)
    for i in range(_F - 1):
        A[0, i + 1] = 1.0
        A[i + 1, 0] = 1.0
    filt = np.eye(_F) - Dg @ (A @ Dg)
    _, U = np.linalg.eigh(filt)
    return U.astype(np.float32)[1:_S + 1, :]


_P_CONST = _make_P()


def _sc_gather_body(neigh_hbm, table_hbm, rows_hbm,
                    idx_v, rows_v, sem_g, sem_w):
    wid = lax.axis_index("s") * _NC + lax.axis_index("c")

    pltpu.sync_copy(neigh_hbm.at[pl.ds(wid * _IDX_ROWS, _IDX_ROWS)], idx_v)

    gets = [
        pltpu.async_copy(table_hbm.at[idx_v.at[q]],
                         rows_v.at[pl.ds(q * 128, 128)], sem_g)
        for q in range(_IDX_ROWS)
    ]

    blk = wid // (_NODE_BLK // _NPW)
    col0 = (wid % (_NODE_BLK // _NPW)) * _NPW * _D

    def _put(t):
        return pltpu.async_copy(
            rows_v.at[pl.ds(t * _NPW, _NPW)].reshape(1, _NPW * _D),
            rows_hbm.at[pl.ds(blk * _S + t, 1), pl.ds(col0, _NPW * _D)],
            sem_w)

    gets[0].wait()
    gets[1].wait()
    puts = [_put(t) for t in range(8)]
    gets[2].wait()
    gets[3].wait()
    puts += [_put(t) for t in range(8, _S)]
    for c in puts:
        c.wait()


@functools.partial(
    pl.kernel,
    out_type=jax.ShapeDtypeStruct((_B // _NODE_BLK * _S, _NODE_BLK * _D),
                                  jnp.float32),
    mesh=plsc.VectorSubcoreMesh(core_axis_name="c", subcore_axis_name="s"),
    scratch_types=[
        pltpu.VMEM((_IDX_ROWS, 128), jnp.int32),
        pltpu.VMEM((_RPW, _D), jnp.float32),
        pltpu.SemaphoreType.DMA,
        pltpu.SemaphoreType.DMA,
    ],
)
def _sc_gather(neigh_hbm, table_hbm, rows_hbm,
               idx_v, rows_v, sem_g, sem_w):
    _sc_gather_body(neigh_hbm, table_hbm, rows_hbm,
                    idx_v, rows_v, sem_g, sem_w)


def _spectral_body(x_ref, p_ref, w_ref, m_ref, o_ref):
    xm = x_ref[...]
    p = p_ref[...]
    w2 = lax.dot_general(p, xm, (((0,), (0,)), ((), ())),
                         precision=lax.Precision.DEFAULT,
                         preferred_element_type=jnp.float32)
    m3 = w_ref[...] * w2
    tf = lax.dot_general(p, m3, (((1,), (0,)), ((), ())),
                         precision=lax.Precision.DEFAULT,
                         preferred_element_type=jnp.float32)
    o_ref[...] = lax.dot_general(m_ref[...], tf, (((1,), (0,)), ((), ())),
                                 precision=lax.Precision.DEFAULT,
                                 preferred_element_type=jnp.float32)


def _bn_body(x_ref, gamma_ref, beta_ref, o_ref):
    x = x_ref[...]
    mean = jnp.mean(x, axis=0, keepdims=True)
    xc = x - mean
    var = jnp.mean(xc * xc, axis=0, keepdims=True)
    inv = lax.rsqrt(var + 1e-5)
    o_ref[...] = xc * (inv * gamma_ref[...]) + beta_ref[...]


def kernel(nodes, batch_neighbors, feat_table, weight, bn_gamma, bn_beta):
    del nodes
    neigh = (batch_neighbors.reshape(_NW, _NPW, _S)
             .transpose(0, 2, 1)
             .reshape(_NW * _IDX_ROWS, 128))
    x = _sc_gather(neigh, feat_table)

    bf_flat = pl.pallas_call(
        _spectral_body,
        grid=(_B // _NODE_BLK,),
        in_specs=[
            pl.BlockSpec((_S, _COL_BLK), lambda i: (i, 0)),
            pl.BlockSpec((_S, _F), lambda i: (0, 0)),
            pl.BlockSpec((_F, 1), lambda i: (0, 0)),
            pl.BlockSpec((1, _S), lambda i: (0, 0)),
        ],
        out_specs=pl.BlockSpec((1, _COL_BLK), lambda i: (0, i)),
        out_shape=jax.ShapeDtypeStruct((1, _B * _D), jnp.float32),
    )(x, jnp.asarray(_P_CONST), weight,
      jnp.full((1, _S), 1.0 / _S, jnp.float32))

    bf = bf_flat.reshape(_B, _D)
    return pl.pallas_call(
        _bn_body,
        out_shape=jax.ShapeDtypeStruct((_B, _D), jnp.float32),
    )(bf, bn_gamma.reshape(1, _D), bn_beta.reshape(1, _D))


# speedup vs baseline: 1.1937x; 1.0591x over previous
import functools

import jax
import jax.numpy as jnp
import numpy as np
from jax import lax
from jax.experimental import pallas as pl
from jax.experimental.pallas import tpu as pltpu
from jax.experimental.pallas import tpu_sc as plsc

_F = 18
_S = _F - 2
_D = 128
_B = 1024

_NC = 2
_NS = 16
_NW = _NC * _NS
_NPW = _B // _NW
_RPW = _NPW * _S
_IDX_ROWS = _RPW // 128

_NODE_BLK = 256
_COL_BLK = _NODE_BLK * _D


def _make_P() -> np.ndarray:
    A = np.zeros((_F, _F))
    Dg = np.eye(_F)
    Dg[0, 0] = (_F - 1) ** (-0.5)
    for i in range(_F - 1):
        A[0, i + 1] = 1.0
        A[i + 1, 0] = 1.0
    filt = np.eye(_F) - Dg @ (A @ Dg)
    _, U = np.linalg.eigh(filt)
    return U.astype(np.float32)[1:_S + 1, :]


_P_CONST = _make_P()


def _sc_gather_body(neigh_hbm, table_hbm, rows_hbm,
                    idx_v, rows_v, sem_g, sem_w):
    wid = lax.axis_index("s") * _NC + lax.axis_index("c")

    pltpu.sync_copy(neigh_hbm.at[pl.ds(wid * _IDX_ROWS, _IDX_ROWS)], idx_v)

    gets = [
        pltpu.async_copy(table_hbm.at[idx_v.at[q]],
                         rows_v.at[pl.ds(q * 128, 128)], sem_g)
        for q in range(_IDX_ROWS)
    ]

    blk = wid // (_NODE_BLK // _NPW)
    col0 = (wid % (_NODE_BLK // _NPW)) * _NPW * _D

    def _put(t):
        return pltpu.async_copy(
            rows_v.at[pl.ds(t * _NPW, _NPW)].reshape(1, _NPW * _D),
            rows_hbm.at[pl.ds(blk * _S + t, 1), pl.ds(col0, _NPW * _D)],
            sem_w)

    gets[0].wait()
    gets[1].wait()
    puts = [_put(t) for t in range(8)]
    gets[2].wait()
    gets[3].wait()
    puts += [_put(t) for t in range(8, _S)]
    for c in puts:
        c.wait()


@functools.partial(
    pl.kernel,
    out_type=jax.ShapeDtypeStruct((_B // _NODE_BLK * _S, _NODE_BLK * _D),
                                  jnp.float32),
    mesh=plsc.VectorSubcoreMesh(core_axis_name="c", subcore_axis_name="s"),
    scratch_types=[
        pltpu.VMEM((_IDX_ROWS, 128), jnp.int32),
        pltpu.VMEM((_RPW, _D), jnp.float32),
        pltpu.SemaphoreType.DMA,
        pltpu.SemaphoreType.DMA,
    ],
)
def _sc_gather(neigh_hbm, table_hbm, rows_hbm,
               idx_v, rows_v, sem_g, sem_w):
    _sc_gather_body(neigh_hbm, table_hbm, rows_hbm,
                    idx_v, rows_v, sem_g, sem_w)


def _spectral_body(x_ref, p_ref, w_ref, m_ref, o_ref):
    xm = x_ref[...]
    p = p_ref[...]
    w2 = lax.dot_general(p, xm, (((0,), (0,)), ((), ())),
                         precision=lax.Precision.DEFAULT,
                         preferred_element_type=jnp.float32)
    m3 = w_ref[...] * w2
    tf = lax.dot_general(p, m3, (((1,), (0,)), ((), ())),
                         precision=lax.Precision.DEFAULT,
                         preferred_element_type=jnp.float32)
    o_ref[...] = lax.dot_general(m_ref[...], tf, (((1,), (0,)), ((), ())),
                                 precision=lax.Precision.DEFAULT,
                                 preferred_element_type=jnp.float32)


def _bn_body(x_ref, gamma_ref, beta_ref, o_ref):
    x = x_ref[...]
    mean = jnp.mean(x, axis=0, keepdims=True)
    xc = x - mean
    var = jnp.mean(xc * xc, axis=0, keepdims=True)
    inv = lax.rsqrt(var + 1e-5)
    o_ref[...] = xc * (inv * gamma_ref[...]) + beta_ref[...]


def kernel(nodes, batch_neighbors, feat_table, weight, bn_gamma, bn_beta):
    del nodes
    neigh = (batch_neighbors.reshape(_NW, _NPW, _S)
             .transpose(0, 2, 1)
             .reshape(_NW * _IDX_ROWS, 128))
    x = _sc_gather(neigh, feat_table)

    bf_flat = pl.pallas_call(
        _spectral_body,
        grid=(_B // _NODE_BLK,),
        in_specs=[
            pl.BlockSpec((_S, _COL_BLK), lambda i: (i, 0)),
            pl.BlockSpec((_S, _F), lambda i: (0, 0)),
            pl.BlockSpec((_F, 1), lambda i: (0, 0)),
            pl.BlockSpec((1, _S), lambda i: (0, 0)),
        ],
        out_specs=pl.BlockSpec((1, _COL_BLK), lambda i: (0, i)),
        out_shape=jax.ShapeDtypeStruct((1, _B * _D), jnp.float32),
    )(x, jnp.asarray(_P_CONST), weight,
      jnp.full((1, _S), 1.0 / _S, jnp.float32))

    bf = bf_flat.reshape(_B, _D)
    return pl.pallas_call(
        _bn_body,
        out_shape=jax.ShapeDtypeStruct((_B, _D), jnp.float32),
    )(bf, bn_gamma.reshape(1, _D), bn_beta.reshape(1, _D))


# speedup vs baseline: 1.2114x; 1.0148x over previous
import functools

import jax
import jax.numpy as jnp
import numpy as np
from jax import lax
from jax.experimental import pallas as pl
from jax.experimental.pallas import tpu as pltpu
from jax.experimental.pallas import tpu_sc as plsc

_F = 18
_S = _F - 2
_D = 128
_B = 1024

_NC = 2
_NS = 16
_NW = _NC * _NS
_NPW = _B // _NW
_RPW = _NPW * _S
_IDX_ROWS = _RPW // 128

_NODE_BLK = 512
_COL_BLK = _NODE_BLK * _D


def _make_P() -> np.ndarray:
    A = np.zeros((_F, _F))
    Dg = np.eye(_F)
    Dg[0, 0] = (_F - 1) ** (-0.5)
    for i in range(_F - 1):
        A[0, i + 1] = 1.0
        A[i + 1, 0] = 1.0
    filt = np.eye(_F) - Dg @ (A @ Dg)
    _, U = np.linalg.eigh(filt)
    return U.astype(np.float32)[1:_S + 1, :]


_P_CONST = _make_P()


def _sc_gather_body(neigh_hbm, table_hbm, rows_hbm,
                    idx_v, rows_v, sem_g, sem_w):
    wid = lax.axis_index("s") * _NC + lax.axis_index("c")

    pltpu.sync_copy(neigh_hbm.at[pl.ds(wid * _IDX_ROWS, _IDX_ROWS)], idx_v)

    gets = [
        pltpu.async_copy(table_hbm.at[idx_v.at[q]],
                         rows_v.at[pl.ds(q * 128, 128)], sem_g)
        for q in range(_IDX_ROWS)
    ]

    blk = wid // (_NODE_BLK // _NPW)
    col0 = (wid % (_NODE_BLK // _NPW)) * _NPW * _D

    def _put(t):
        return pltpu.async_copy(
            rows_v.at[pl.ds(t * _NPW, _NPW)].reshape(1, _NPW * _D),
            rows_hbm.at[pl.ds(blk * _S + t, 1), pl.ds(col0, _NPW * _D)],
            sem_w)

    gets[0].wait()
    gets[1].wait()
    puts = [_put(t) for t in range(8)]
    gets[2].wait()
    gets[3].wait()
    puts += [_put(t) for t in range(8, _S)]
    for c in puts:
        c.wait()


@functools.partial(
    pl.kernel,
    out_type=jax.ShapeDtypeStruct((_B // _NODE_BLK * _S, _NODE_BLK * _D),
                                  jnp.float32),
    mesh=plsc.VectorSubcoreMesh(core_axis_name="c", subcore_axis_name="s"),
    scratch_types=[
        pltpu.VMEM((_IDX_ROWS, 128), jnp.int32),
        pltpu.VMEM((_RPW, _D), jnp.float32),
        pltpu.SemaphoreType.DMA,
        pltpu.SemaphoreType.DMA,
    ],
)
def _sc_gather(neigh_hbm, table_hbm, rows_hbm,
               idx_v, rows_v, sem_g, sem_w):
    _sc_gather_body(neigh_hbm, table_hbm, rows_hbm,
                    idx_v, rows_v, sem_g, sem_w)


def _spectral_body(x_ref, p_ref, w_ref, m_ref, o_ref):
    xm = x_ref[...]
    p = p_ref[...]
    w2 = lax.dot_general(p, xm, (((0,), (0,)), ((), ())),
                         precision=lax.Precision.DEFAULT,
                         preferred_element_type=jnp.float32)
    m3 = w_ref[...] * w2
    tf = lax.dot_general(p, m3, (((1,), (0,)), ((), ())),
                         precision=lax.Precision.DEFAULT,
                         preferred_element_type=jnp.float32)
    o_ref[...] = lax.dot_general(m_ref[...], tf, (((1,), (0,)), ((), ())),
                                 precision=lax.Precision.DEFAULT,
                                 preferred_element_type=jnp.float32)


def _bn_body(x_ref, gamma_ref, beta_ref, o_ref):
    x = x_ref[...]
    mean = jnp.mean(x, axis=0, keepdims=True)
    xc = x - mean
    var = jnp.mean(xc * xc, axis=0, keepdims=True)
    inv = lax.rsqrt(var + 1e-5)
    o_ref[...] = xc * (inv * gamma_ref[...]) + beta_ref[...]


def kernel(nodes, batch_neighbors, feat_table, weight, bn_gamma, bn_beta):
    del nodes
    neigh = (batch_neighbors.reshape(_NW, _NPW, _S)
             .transpose(0, 2, 1)
             .reshape(_NW * _IDX_ROWS, 128))
    x = _sc_gather(neigh, feat_table)

    bf_flat = pl.pallas_call(
        _spectral_body,
        grid=(_B // _NODE_BLK,),
        in_specs=[
            pl.BlockSpec((_S, _COL_BLK), lambda i: (i, 0)),
            pl.BlockSpec((_S, _F), lambda i: (0, 0)),
            pl.BlockSpec((_F, 1), lambda i: (0, 0)),
            pl.BlockSpec((1, _S), lambda i: (0, 0)),
        ],
        out_specs=pl.BlockSpec((1, _COL_BLK), lambda i: (0, i)),
        out_shape=jax.ShapeDtypeStruct((1, _B * _D), jnp.float32),
    )(x, jnp.asarray(_P_CONST), weight,
      jnp.full((1, _S), 1.0 / _S, jnp.float32))

    bf = bf_flat.reshape(_B, _D)
    return pl.pallas_call(
        _bn_body,
        out_shape=jax.ShapeDtypeStruct((_B, _D), jnp.float32),
    )(bf, bn_gamma.reshape(1, _D), bn_beta.reshape(1, _D))
